# baseline (speedup 1.0000x reference)
import functools

import jax
import jax.numpy as jnp
from jax import lax
from jax.experimental import pallas as pl
from jax.experimental.pallas import tpu as pltpu
from jax.experimental.pallas import tpu_sc as plsc

B, L, V = 64, 4, 100000
R = B * L
CW = 4096
NC = (V + CW - 1) // CW
TAIL = V - (NC - 1) * CW
K = 2
NCI = NC - 1 - K

_KS0 = 0
_KS1 = 42
_KS2 = 0x1BD11BDA ^ _KS0 ^ _KS1

_NEG_INF = float("-inf")
_TINY = 1.1754943508222875e-38


def _rotl(x, r):
    return lax.shift_left(x, r) | lax.shift_right_logical(x, 32 - r)


def _threefry_bits(f):
    v0 = jnp.zeros_like(f) + jnp.int32(_KS0)
    v1 = f + jnp.int32(_KS1)

    def rounds(v0, v1, rots):
        for r in rots:
            v0 = v0 + v1
            v1 = _rotl(v1, r) ^ v0
        return v0, v1

    r0 = (13, 15, 26, 6)
    r1 = (17, 29, 16, 24)
    v0, v1 = rounds(v0, v1, r0)
    v0 += jnp.int32(_KS1); v1 += jnp.int32(_KS2 + 1)
    v0, v1 = rounds(v0, v1, r1)
    v0 += jnp.int32(_KS2); v1 += jnp.int32(_KS0 + 2)
    v0, v1 = rounds(v0, v1, r0)
    v0 += jnp.int32(_KS0); v1 += jnp.int32(_KS1 + 3)
    v0, v1 = rounds(v0, v1, r1)
    v0 += jnp.int32(_KS1); v1 += jnp.int32(_KS2 + 4)
    v0, v1 = rounds(v0, v1, r0)
    v0 += jnp.int32(_KS2); v1 += jnp.int32(_KS0 + 5)
    return v0 ^ v1


def _bits_to_y(bits, x):
    fb = lax.shift_right_logical(bits, 9) | jnp.int32(0x3F800000)
    u = lax.bitcast_convert_type(fb, jnp.float32) - jnp.float32(1.0)
    u = jnp.maximum(u, jnp.float32(_TINY))
    return -jnp.log(-jnp.log(u)) + x


def _gumbel_y(f, x):
    return _bits_to_y(_threefry_bits(f), x)


def _sc_bits_fn():
    mesh = plsc.VectorSubcoreMesh(core_axis_name="c", subcore_axis_name="s")
    info = plsc.get_sparse_core_info()
    ncore, nsub, ln = info.num_cores, info.num_subcores, info.num_lanes
    nw = ncore * nsub
    rpw = R // nw

    @functools.partial(
        pl.kernel,
        mesh=mesh,
        out_type=jax.ShapeDtypeStruct((K, R, CW), jnp.int32),
        scratch_types=[
            pltpu.VMEM((rpw, CW), jnp.int32),
            pltpu.SemaphoreType.DMA,
        ],
    )
    def sc_bits(out_hbm, buf, sem):
        wid = lax.axis_index("s") * ncore + lax.axis_index("c")
        base_row = wid * rpw
        lane = lax.iota(jnp.int32, ln)

        def chunk(k, carry):
            col0 = (NCI + k) * CW

            def body(j, c2):
                col = col0 + j * ln
                for i in range(rpw):
                    f = (base_row + i) * V + col + lane
                    buf[i, pl.ds(j * ln, ln)] = _threefry_bits(f)
                return c2

            lax.fori_loop(0, CW // ln, body, 0)
            pltpu.sync_copy(buf, out_hbm.at[k, pl.ds(base_row, rpw)])
            return carry

        lax.fori_loop(0, K, chunk, 0)

    return sc_bits


def _tc1(x_ref, ry_out, rc_out, ry_ref, rc_ref, fb_ref):
    pc = pl.program_id(0)

    @pl.when(pc == 0)
    def _first():
        row = lax.broadcasted_iota(jnp.int32, (R, CW), 0)
        lane = lax.broadcasted_iota(jnp.int32, (R, CW), 1)
        f0 = row * V + lane
        fb_ref[...] = f0
        ry_ref[...] = _gumbel_y(f0, x_ref[...])
        rc_ref[...] = jnp.zeros((R, CW), jnp.int32)

    @pl.when(pc > 0)
    def _main():
        y = _gumbel_y(fb_ref[...] + pc * CW, x_ref[...])
        ry = ry_ref[...]
        upd = y > ry
        ry_ref[...] = jnp.where(upd, y, ry)
        rc_ref[...] = jnp.where(upd, pc, rc_ref[...])

    @pl.when(pc == NCI - 1)
    def _flush():
        ry_out[...] = ry_ref[...]
        rc_out[...] = rc_ref[...]


def _tc2(x_ref, bits_ref, ry0_ref, rc0_ref, out_ref, ry_ref, rc_ref):
    pc = pl.program_id(0)

    @pl.when(pc == 0)
    def _first():
        y = _bits_to_y(bits_ref[0], x_ref[...])
        ry = ry0_ref[...]
        upd = y > ry
        ry_ref[...] = jnp.where(upd, y, ry)
        rc_ref[...] = jnp.where(upd, NCI, rc0_ref[...])

    @pl.when(jnp.logical_and(pc > 0, pc < K))
    def _mid():
        y = _bits_to_y(bits_ref[0], x_ref[...])
        ry = ry_ref[...]
        upd = y > ry
        ry_ref[...] = jnp.where(upd, y, ry)
        rc_ref[...] = jnp.where(upd, NCI + pc, rc_ref[...])

    @pl.when(pc == K)
    def _tail_and_finish():
        row = lax.broadcasted_iota(jnp.int32, (R, CW), 0)
        lane = lax.broadcasted_iota(jnp.int32, (R, CW), 1)
        f = row * V + (NC - 1) * CW + lane
        y = _gumbel_y(f, x_ref[...])
        y = jnp.where(lane < TAIL, y, _NEG_INF)
        ry = ry_ref[...]
        upd = y > ry
        ry = jnp.where(upd, y, ry)
        rc = jnp.where(upd, NC - 1, rc_ref[...])
        col = rc * CW + lane
        m = jnp.max(ry, axis=1, keepdims=True)
        idx = jnp.min(jnp.where(ry == m, col, jnp.int32(V)), axis=1)
        out_ref[...] = idx.reshape(1, 1, R)


def kernel(logits):
    x = logits.reshape(R, V)

    bits = _sc_bits_fn()()

    ry0, rc0 = pl.pallas_call(
        _tc1,
        grid=(NCI,),
        in_specs=[pl.BlockSpec((R, CW), lambda c: (0, c))],
        out_specs=[
            pl.BlockSpec((R, CW), lambda c: (0, 0)),
            pl.BlockSpec((R, CW), lambda c: (0, 0)),
        ],
        out_shape=[
            jax.ShapeDtypeStruct((R, CW), jnp.float32),
            jax.ShapeDtypeStruct((R, CW), jnp.int32),
        ],
        scratch_shapes=[
            pltpu.VMEM((R, CW), jnp.float32),
            pltpu.VMEM((R, CW), jnp.int32),
            pltpu.VMEM((R, CW), jnp.int32),
        ],
    )(x)

    out = pl.pallas_call(
        _tc2,
        grid=(K + 1,),
        in_specs=[
            pl.BlockSpec((R, CW), lambda c: (0, NCI + c)),
            pl.BlockSpec((1, R, CW), lambda c: (jnp.minimum(c, K - 1), 0, 0)),
            pl.BlockSpec((R, CW), lambda c: (0, 0)),
            pl.BlockSpec((R, CW), lambda c: (0, 0)),
        ],
        out_specs=pl.BlockSpec((1, 1, R), lambda c: (0, 0, 0)),
        out_shape=jax.ShapeDtypeStruct((1, 1, R), jnp.int32),
        scratch_shapes=[
            pltpu.VMEM((R, CW), jnp.float32),
            pltpu.VMEM((R, CW), jnp.int32),
        ],
    )(x, bits, ry0, rc0)

    return out.reshape(B, L)


# speedup vs baseline: 1.9012x; 1.9012x over previous
import jax
import jax.numpy as jnp
from jax.experimental import pallas as pl
from jax.experimental.pallas import tpu as pltpu

B, L, V = 64, 4, 100000
R = B * L
CW = 4096
NC = (V + CW - 1) // CW
TAIL = V - (NC - 1) * CW

_KS0 = 0
_KS1 = 42
_KS2 = 0x1BD11BDA ^ _KS0 ^ _KS1

_NEG_INF = float("-inf")
_TINY = 1.1754943508222875e-38


def _rotl(x, r):
    return jax.lax.shift_left(x, r) | jax.lax.shift_right_logical(x, 32 - r)


def _threefry_bits(f):
    v0 = jnp.zeros_like(f) + jnp.int32(_KS0)
    v1 = f + jnp.int32(_KS1)

    def rounds(v0, v1, rots):
        for r in rots:
            v0 = v0 + v1
            v1 = _rotl(v1, r) ^ v0
        return v0, v1

    r0 = (13, 15, 26, 6)
    r1 = (17, 29, 16, 24)
    v0, v1 = rounds(v0, v1, r0)
    v0 += jnp.int32(_KS1); v1 += jnp.int32(_KS2 + 1)
    v0, v1 = rounds(v0, v1, r1)
    v0 += jnp.int32(_KS2); v1 += jnp.int32(_KS0 + 2)
    v0, v1 = rounds(v0, v1, r0)
    v0 += jnp.int32(_KS0); v1 += jnp.int32(_KS1 + 3)
    v0, v1 = rounds(v0, v1, r1)
    v0 += jnp.int32(_KS1); v1 += jnp.int32(_KS2 + 4)
    v0, v1 = rounds(v0, v1, r0)
    v0 += jnp.int32(_KS2); v1 += jnp.int32(_KS0 + 5)
    return v0 ^ v1


def _gumbel_y(f, x):
    bits = _threefry_bits(f)
    fb = jax.lax.shift_right_logical(bits, 9) | jnp.int32(0x3F800000)
    u = jax.lax.bitcast_convert_type(fb, jnp.float32) - jnp.float32(1.0)
    u = jnp.maximum(u, jnp.float32(_TINY))
    return -jnp.log(-jnp.log(u)) + x


def _kernel(x_ref, out_ref, ry_ref, rc_ref, fb_ref):
    pc = pl.program_id(0)

    @pl.when(pc == 0)
    def _first():
        row = jax.lax.broadcasted_iota(jnp.int32, (R, CW), 0)
        lane = jax.lax.broadcasted_iota(jnp.int32, (R, CW), 1)
        f0 = row * V + lane
        fb_ref[...] = f0
        y = _gumbel_y(f0 + (NC - 1) * CW, x_ref[...])
        ry_ref[...] = jnp.where(lane < TAIL, y, _NEG_INF)
        rc_ref[...] = jnp.full((R, CW), NC - 1, jnp.int32)

    @pl.when(pc > 0)
    def _main():
        cid = pc - 1
        y = _gumbel_y(fb_ref[...] + cid * CW, x_ref[...])
        ry = ry_ref[...]
        upd = y > ry
        ry_ref[...] = jnp.where(upd, y, ry)
        rc_ref[...] = jnp.where(upd, cid, rc_ref[...])

    @pl.when(pc == NC - 1)
    def _finish():
        lane = jax.lax.broadcasted_iota(jnp.int32, (R, CW), 1)
        ry = ry_ref[...]
        col = rc_ref[...] * CW + lane
        m = jnp.max(ry, axis=1, keepdims=True)
        idx = jnp.min(jnp.where(ry == m, col, jnp.int32(V)), axis=1)
        out_ref[...] = idx.reshape(1, 1, R)


def kernel(logits):
    x = logits.reshape(R, V)
    out = pl.pallas_call(
        _kernel,
        grid=(NC,),
        in_specs=[pl.BlockSpec((R, CW), lambda c: (0, (c + NC - 1) % NC))],
        out_specs=pl.BlockSpec((1, 1, R), lambda c: (0, 0, 0)),
        out_shape=jax.ShapeDtypeStruct((1, 1, R), jnp.int32),
        scratch_shapes=[
            pltpu.VMEM((R, CW), jnp.float32),
            pltpu.VMEM((R, CW), jnp.int32),
            pltpu.VMEM((R, CW), jnp.int32),
        ],
    )(x)
    return out.reshape(B, L)


# speedup vs baseline: 1.9055x; 1.0023x over previous
import jax
import jax.numpy as jnp
from jax.experimental import pallas as pl
from jax.experimental.pallas import tpu as pltpu

B, L, V = 64, 4, 100000
R = B * L
CW = 4096
NC = (V + CW - 1) // CW
TAIL = V - (NC - 1) * CW

_KS0 = 0
_KS1 = 42
_KS2 = 0x1BD11BDA ^ _KS0 ^ _KS1

_NEG_INF = float("-inf")
_TINY = 1.1754943508222875e-38


def _rotl(x, r):
    return jax.lax.shift_left(x, r) | jax.lax.shift_right_logical(x, 32 - r)


def _threefry_bits(fk):
    v0 = jnp.zeros_like(fk) + jnp.int32(_KS0)
    v1 = fk

    def rounds(v0, v1, rots):
        for r in rots:
            v0 = v0 + v1
            v1 = _rotl(v1, r) ^ v0
        return v0, v1

    r0 = (13, 15, 26, 6)
    r1 = (17, 29, 16, 24)
    v0, v1 = rounds(v0, v1, r0)
    v0 += jnp.int32(_KS1); v1 += jnp.int32(_KS2 + 1)
    v0, v1 = rounds(v0, v1, r1)
    v0 += jnp.int32(_KS2); v1 += jnp.int32(_KS0 + 2)
    v0, v1 = rounds(v0, v1, r0)
    v0 += jnp.int32(_KS0); v1 += jnp.int32(_KS1 + 3)
    v0, v1 = rounds(v0, v1, r1)
    v0 += jnp.int32(_KS1); v1 += jnp.int32(_KS2 + 4)
    v0, v1 = rounds(v0, v1, r0)
    v0 += jnp.int32(_KS2); v1 += jnp.int32(_KS0 + 5)
    return v0 ^ v1


def _gumbel_y(f, x):
    bits = _threefry_bits(f)
    fb = jax.lax.shift_right_logical(bits, 9) | jnp.int32(0x3F800000)
    u = jax.lax.bitcast_convert_type(fb, jnp.float32) - jnp.float32(1.0)
    u = jnp.maximum(u, jnp.float32(_TINY))
    return -jnp.log(-jnp.log(u)) + x


def _kernel(x_ref, out_ref, ry_ref, rc_ref, fb_ref):
    pc = pl.program_id(0)

    @pl.when(pc == 0)
    def _first():
        row = jax.lax.broadcasted_iota(jnp.int32, (R, CW), 0)
        lane = jax.lax.broadcasted_iota(jnp.int32, (R, CW), 1)
        f0 = row * V + lane + _KS1
        fb_ref[...] = f0
        y = _gumbel_y(f0 + (NC - 1) * CW, x_ref[...])
        ry_ref[...] = jnp.where(lane < TAIL, y, _NEG_INF)
        rc_ref[...] = jnp.full((R, CW), NC - 1, jnp.int32)

    @pl.when(pc > 0)
    def _main():
        cid = pc - 1
        y = _gumbel_y(fb_ref[...] + cid * CW, x_ref[...])
        ry = ry_ref[...]
        upd = y > ry
        ry_ref[...] = jnp.where(upd, y, ry)
        rc_ref[...] = jnp.where(upd, cid, rc_ref[...])

    @pl.when(pc == NC - 1)
    def _finish():
        lane = jax.lax.broadcasted_iota(jnp.int32, (R, CW), 1)
        ry = ry_ref[...]
        col = rc_ref[...] * CW + lane
        m = jnp.max(ry, axis=1, keepdims=True)
        idx = jnp.min(jnp.where(ry == m, col, jnp.int32(V)), axis=1)
        out_ref[...] = idx.reshape(1, 1, R)


def kernel(logits):
    x = logits.reshape(R, V)
    out = pl.pallas_call(
        _kernel,
        grid=(NC,),
        in_specs=[pl.BlockSpec((R, CW), lambda c: (0, (c + NC - 1) % NC))],
        out_specs=pl.BlockSpec((1, 1, R), lambda c: (0, 0, 0)),
        out_shape=jax.ShapeDtypeStruct((1, 1, R), jnp.int32),
        scratch_shapes=[
            pltpu.VMEM((R, CW), jnp.float32),
            pltpu.VMEM((R, CW), jnp.int32),
            pltpu.VMEM((R, CW), jnp.int32),
        ],
    )(x)
    return out.reshape(B, L)
